# baseline (speedup 1.0000x reference)
import functools

import jax
import jax.numpy as jnp
from jax import lax
from jax.experimental import pallas as pl
from jax.experimental.pallas import tpu as pltpu
from jax.experimental.pallas import tpu_sc as plsc

N_QUANT = 8
CODEBOOK_SIZE = 8192
CHUNK = 128
BATCH = 16
SEQ = 4096
TOKENS = BATCH * SEQ
ROWS = TOKENS * N_QUANT
NUM_WORKERS = 32
TOK_W = TOKENS // NUM_WORKERS
CT = 32
CROWS = CT * N_QUANT
NCH = TOK_W // CT
IDXR = TOK_W * N_QUANT // 128

_mesh = plsc.VectorSubcoreMesh(core_axis_name="c", subcore_axis_name="s")


@functools.partial(
    pl.kernel,
    mesh=_mesh,
    out_type=jax.ShapeDtypeStruct((BATCH, SEQ, N_QUANT * CHUNK), jnp.float32),
    scratch_types=(
        [pltpu.VMEM((TOK_W * N_QUANT,), jnp.int32)]
        + [pltpu.VMEM((2, 128), jnp.int32)]
        + [pltpu.VMEM((CROWS, CHUNK), jnp.float32)]
        + [pltpu.SemaphoreType.DMA, pltpu.SemaphoreType.DMA]
    ),
    compiler_params=pltpu.CompilerParams(needs_layout_passes=False),
)
def _emb_lookup(tbl_hbm, idx_hbm, out_hbm, idx_raw, idx_perm, rows_v, gsem, osem):
    wid = lax.axis_index("s") * 2 + lax.axis_index("c")
    base = wid * TOK_W * N_QUANT
    b = wid // 2
    t_base = (wid % 2) * TOK_W

    pltpu.sync_copy(idx_hbm.at[pl.ds(base, TOK_W * N_QUANT)], idx_raw)

    lanes = lax.iota(jnp.int32, 16)

    def body(j, carry):
        jchunk = jnp.full((16,), CROWS * j, jnp.int32)
        for g in range(CROWS // 16):
            q = 16 * g + lanes
            s = (q % 32) * 8 + (16 * g) // 32
            vals = plsc.load_gather(idx_raw, [jchunk + s])
            iq = (16 * g) // 32
            vals = vals + iq * CODEBOOK_SIZE
            idx_perm[(16 * g) // 128, pl.ds((16 * g) % 128, 16)] = vals
        pltpu.async_copy(
            tbl_hbm.at[idx_perm.at[0]], rows_v.at[pl.ds(0, 128)], gsem
        ).wait()
        pltpu.async_copy(
            tbl_hbm.at[idx_perm.at[1]], rows_v.at[pl.ds(128, 128)], gsem
        ).wait()
        t0 = t_base + j * CT
        for i in range(N_QUANT):
            pltpu.async_copy(
                rows_v.at[pl.ds(i * CT, CT)],
                out_hbm.at[b, pl.ds(t0, CT), pl.ds(i * CHUNK, CHUNK)],
                osem,
            ).wait()
        return carry

    lax.fori_loop(0, NCH, body, 0)


def kernel(x, tables):
    xf = x.astype(jnp.int32).reshape(TOKENS * N_QUANT)
    tbl = tables.reshape(N_QUANT * CODEBOOK_SIZE, CHUNK)
    return _emb_lookup(tbl, xf)


# speedup vs baseline: 1.4483x; 1.4483x over previous
import functools

import jax
import jax.numpy as jnp
from jax import lax
from jax.experimental import pallas as pl
from jax.experimental.pallas import tpu as pltpu
from jax.experimental.pallas import tpu_sc as plsc

N_QUANT = 8
CODEBOOK_SIZE = 8192
CHUNK = 128
BATCH = 16
SEQ = 4096
TOKENS = BATCH * SEQ
NUM_WORKERS = 32
TOK_W = TOKENS // NUM_WORKERS
CT = 32
CROWS = CT * N_QUANT
NCH = TOK_W // CT

_mesh = plsc.VectorSubcoreMesh(core_axis_name="c", subcore_axis_name="s")


@functools.partial(
    pl.kernel,
    mesh=_mesh,
    out_type=jax.ShapeDtypeStruct((BATCH, SEQ, N_QUANT * CHUNK), jnp.float32),
    scratch_types=(
        [pltpu.VMEM((TOK_W * N_QUANT,), jnp.int32)]
        + [pltpu.VMEM((2, 128), jnp.int32) for _ in range(2)]
        + [pltpu.VMEM((CROWS, CHUNK), jnp.float32) for _ in range(2)]
        + [pltpu.SemaphoreType.DMA for _ in range(4)]
    ),
    compiler_params=pltpu.CompilerParams(needs_layout_passes=False),
)
def _emb_lookup(tbl_hbm, idx_hbm, out_hbm, idx_raw, ipA, ipB, rowsA, rowsB,
                gsemA, gsemB, osemA, osemB):
    wid = lax.axis_index("s") * 2 + lax.axis_index("c")
    base = wid * TOK_W * N_QUANT
    b = wid // 2
    t_base = (wid % 2) * TOK_W

    pltpu.sync_copy(idx_hbm.at[pl.ds(base, TOK_W * N_QUANT)], idx_raw)

    lanes = lax.iota(jnp.int32, 16)

    def build_perm(c, ip):
        jchunk = jnp.full((16,), CROWS * c, jnp.int32)
        for g in range(CROWS // 16):
            q = 16 * g + lanes
            s = (q % CT) * N_QUANT + (16 * g) // CT
            vals = plsc.load_gather(idx_raw, [jchunk + s])
            iq = (16 * g) // CT
            vals = vals + iq * CODEBOOK_SIZE
            ip[(16 * g) // 128, pl.ds((16 * g) % 128, 16)] = vals

    def fire_gathers(ip, rows, gsem):
        h0 = pltpu.async_copy(tbl_hbm.at[ip.at[0]], rows.at[pl.ds(0, 128)], gsem)
        h1 = pltpu.async_copy(tbl_hbm.at[ip.at[1]], rows.at[pl.ds(128, 128)], gsem)
        return h0, h1

    def fire_writebacks(c, rows, osem):
        t0 = t_base + c * CT
        for i in range(N_QUANT):
            pltpu.async_copy(
                rows.at[pl.ds(i * CT, CT)],
                out_hbm.at[b, pl.ds(t0, CT), pl.ds(i * CHUNK, CHUNK)],
                osem,
            )

    def drain(sem, rows):
        pltpu.make_async_copy(tbl_hbm.at[pl.ds(0, CROWS)], rows, sem).wait()

    build_perm(0, ipA)

    def body(g, carry):
        a = 2 * g
        @pl.when(g > 0)
        def _():
            drain(osemA, rowsA)
        ga = fire_gathers(ipA, rowsA, gsemA)
        build_perm(a + 1, ipB)
        @pl.when(g > 0)
        def _():
            drain(osemB, rowsB)
        gb = fire_gathers(ipB, rowsB, gsemB)
        ga[0].wait()
        ga[1].wait()
        fire_writebacks(a, rowsA, osemA)

        @pl.when(g < NCH // 2 - 1)
        def _():
            build_perm(a + 2, ipA)
        gb[0].wait()
        gb[1].wait()
        fire_writebacks(a + 1, rowsB, osemB)
        return carry

    lax.fori_loop(0, NCH // 2, body, 0)
    drain(osemA, rowsA)
    drain(osemB, rowsB)


def kernel(x, tables):
    xf = x.astype(jnp.int32).reshape(TOKENS * N_QUANT)
    tbl = tables.reshape(N_QUANT * CODEBOOK_SIZE, CHUNK)
    return _emb_lookup(tbl, xf)


# speedup vs baseline: 1.4566x; 1.0057x over previous
import functools

import jax
import jax.numpy as jnp
from jax import lax
from jax.experimental import pallas as pl
from jax.experimental.pallas import tpu as pltpu
from jax.experimental.pallas import tpu_sc as plsc

N_QUANT = 8
CODEBOOK_SIZE = 8192
CHUNK = 128
BATCH = 16
SEQ = 4096
TOKENS = BATCH * SEQ
NUM_WORKERS = 32
TOK_W = TOKENS // NUM_WORKERS
CT = 32
CROWS = CT * N_QUANT
NCH = TOK_W // CT

_mesh = plsc.VectorSubcoreMesh(core_axis_name="c", subcore_axis_name="s")


@functools.partial(
    pl.kernel,
    mesh=_mesh,
    out_type=jax.ShapeDtypeStruct((BATCH, SEQ, N_QUANT * CHUNK), jnp.float32),
    scratch_types=(
        [pltpu.VMEM((TOK_W * N_QUANT,), jnp.int32)]
        + [pltpu.VMEM((N_QUANT, 128), jnp.int32) for _ in range(2)]
        + [pltpu.VMEM((CROWS, CHUNK), jnp.float32) for _ in range(2)]
        + [pltpu.SemaphoreType.DMA for _ in range(4)]
    ),
    compiler_params=pltpu.CompilerParams(needs_layout_passes=False),
)
def _emb_lookup(tbl_hbm, x_hbm, out_hbm, idx_raw, ipA, ipB, rowsA, rowsB,
                gsemA, gsemB, osemA, osemB):
    wid = lax.axis_index("s") * 2 + lax.axis_index("c")
    b = wid // 2
    t_base = (wid % 2) * TOK_W

    pltpu.sync_copy(x_hbm.at[pl.ds(wid * TOK_W * N_QUANT, TOK_W * N_QUANT)], idx_raw)

    lanes = lax.iota(jnp.int32, 16)

    def build_perm(c, ip):
        jchunk = jnp.full((16,), CROWS * c, jnp.int32)
        for g in range(CROWS // 16):
            iq = 16 * g // CT
            tl0 = (16 * g) % CT
            s = (tl0 + lanes) * N_QUANT + iq
            vals = plsc.load_gather(idx_raw, [jchunk + s])
            ip[iq, pl.ds(tl0, 16)] = vals

    def fire_gathers(ip, rows, gsem):
        for i in range(N_QUANT):
            pltpu.async_copy(
                tbl_hbm.at[i].at[ip.at[i, pl.ds(0, CT)]],
                rows.at[pl.ds(i * CT, CT)],
                gsem,
            )

    def drain_gathers(rows, gsem):
        pltpu.make_async_copy(
            tbl_hbm.at[0].at[pl.ds(0, CROWS)], rows, gsem
        ).wait()

    def fire_writebacks(c, rows, osem):
        t0 = t_base + c * CT
        for i in range(N_QUANT):
            pltpu.async_copy(
                rows.at[pl.ds(i * CT, CT)],
                out_hbm.at[b, pl.ds(t0, CT), pl.ds(i * CHUNK, CHUNK)],
                osem,
            )

    def drain_writebacks(rows, osem):
        pltpu.make_async_copy(
            tbl_hbm.at[0].at[pl.ds(0, CROWS)], rows, osem
        ).wait()

    build_perm(0, ipA)

    def body(g, carry):
        a = 2 * g
        @pl.when(g > 0)
        def _():
            drain_writebacks(rowsA, osemA)
        fire_gathers(ipA, rowsA, gsemA)
        build_perm(a + 1, ipB)
        @pl.when(g > 0)
        def _():
            drain_writebacks(rowsB, osemB)
        fire_gathers(ipB, rowsB, gsemB)
        drain_gathers(rowsA, gsemA)
        fire_writebacks(a, rowsA, osemA)

        @pl.when(g < NCH // 2 - 1)
        def _():
            build_perm(a + 2, ipA)
        drain_gathers(rowsB, gsemB)
        fire_writebacks(a + 1, rowsB, osemB)
        return carry

    lax.fori_loop(0, NCH // 2, body, 0)
    drain_writebacks(rowsA, osemA)
    drain_writebacks(rowsB, osemB)


def kernel(x, tables):
    xf = x.astype(jnp.int32).reshape(TOKENS * N_QUANT)
    return _emb_lookup(tables, xf)


# speedup vs baseline: 1.5415x; 1.0583x over previous
import functools

import jax
import jax.numpy as jnp
from jax import lax
from jax.experimental import pallas as pl
from jax.experimental.pallas import tpu as pltpu
from jax.experimental.pallas import tpu_sc as plsc

N_QUANT = 8
CODEBOOK_SIZE = 8192
CHUNK = 128
BATCH = 16
SEQ = 4096
TOKENS = BATCH * SEQ
NUM_WORKERS = 32
TOK_W = TOKENS // NUM_WORKERS
CT = 32
CROWS = CT * N_QUANT
BT = 128
NBLK = TOK_W // BT

_mesh = plsc.VectorSubcoreMesh(core_axis_name="c", subcore_axis_name="s")


@functools.partial(
    pl.kernel,
    mesh=_mesh,
    out_type=jax.ShapeDtypeStruct((BATCH, SEQ, N_QUANT * CHUNK), jnp.float32),
    scratch_types=(
        [pltpu.VMEM((BT, N_QUANT), jnp.int32) for _ in range(2)]
        + [pltpu.VMEM((N_QUANT, 128), jnp.int32) for _ in range(2)]
        + [pltpu.VMEM((CROWS, CHUNK), jnp.float32) for _ in range(2)]
        + [pltpu.SemaphoreType.DMA for _ in range(6)]
    ),
    compiler_params=pltpu.CompilerParams(needs_layout_passes=False),
)
def _emb_lookup(tbl_hbm, x_hbm, out_hbm, xsA, xsB, ipA, ipB, rowsA, rowsB,
                xsemA, xsemB, gsemA, gsemB, osemA, osemB):
    wid = lax.axis_index("s") * 2 + lax.axis_index("c")
    b = wid // 2
    t_base = (wid % 2) * TOK_W

    lanes = lax.iota(jnp.int32, 16)

    def stage_x(blk, xs, xsem):
        pltpu.async_copy(x_hbm.at[b, pl.ds(t_base + blk * BT, BT)], xs, xsem)

    def drain_x(xs, xsem):
        pltpu.make_async_copy(x_hbm.at[0, pl.ds(0, BT)], xs, xsem).wait()

    def build_perm(k, xs, ip):
        for g in range(CROWS // 16):
            iq = 16 * g // CT
            tl0 = (16 * g) % CT
            row = k * CT + tl0 + lanes
            col = jnp.full((16,), iq, jnp.int32)
            vals = plsc.load_gather(xs, [row, col])
            ip[iq, pl.ds(tl0, 16)] = vals

    def fire_gathers(ip, rows, gsem):
        for i in range(N_QUANT):
            pltpu.async_copy(
                tbl_hbm.at[i].at[ip.at[i, pl.ds(0, CT)]],
                rows.at[pl.ds(i * CT, CT)],
                gsem,
            )

    def drain_gathers(rows, gsem):
        pltpu.make_async_copy(
            tbl_hbm.at[0].at[pl.ds(0, CROWS)], rows, gsem
        ).wait()

    def fire_writebacks(c, rows, osem):
        t0 = t_base + c * CT
        for i in range(N_QUANT):
            pltpu.async_copy(
                rows.at[pl.ds(i * CT, CT)],
                out_hbm.at[b, pl.ds(t0, CT), pl.ds(i * CHUNK, CHUNK)],
                osem,
            )

    def drain_writebacks(rows, osem):
        pltpu.make_async_copy(
            tbl_hbm.at[0].at[pl.ds(0, CROWS)], rows, osem
        ).wait()

    def process_block(blk, xs):
        for k in range(0, 4, 2):
            a = 4 * blk + k
            build_perm(k, xs, ipA)

            @pl.when(a > 0)
            def _():
                drain_writebacks(rowsA, osemA)
            fire_gathers(ipA, rowsA, gsemA)
            build_perm(k + 1, xs, ipB)

            @pl.when(a > 0)
            def _():
                drain_writebacks(rowsB, osemB)
            fire_gathers(ipB, rowsB, gsemB)
            drain_gathers(rowsA, gsemA)
            fire_writebacks(a, rowsA, osemA)
            drain_gathers(rowsB, gsemB)
            fire_writebacks(a + 1, rowsB, osemB)

    stage_x(0, xsA, xsemA)
    drain_x(xsA, xsemA)

    def body(g2, carry):
        blk = 2 * g2
        stage_x(blk + 1, xsB, xsemB)
        process_block(blk, xsA)
        drain_x(xsB, xsemB)

        @pl.when(g2 < NBLK // 2 - 1)
        def _():
            stage_x(blk + 2, xsA, xsemA)
        process_block(blk + 1, xsB)

        @pl.when(g2 < NBLK // 2 - 1)
        def _():
            drain_x(xsA, xsemA)
        return carry

    lax.fori_loop(0, NBLK // 2, body, 0)
    drain_writebacks(rowsA, osemA)
    drain_writebacks(rowsB, osemB)


def kernel(x, tables):
    return _emb_lookup(tables, x.astype(jnp.int32))


# speedup vs baseline: 1.5741x; 1.0211x over previous
import functools

import jax
import jax.numpy as jnp
from jax import lax
from jax.experimental import pallas as pl
from jax.experimental.pallas import tpu as pltpu
from jax.experimental.pallas import tpu_sc as plsc

N_QUANT = 8
CODEBOOK_SIZE = 8192
CHUNK = 128
BATCH = 16
SEQ = 4096
TOKENS = BATCH * SEQ
NUM_WORKERS = 32
TOK_W = TOKENS // NUM_WORKERS
CT = 32
CROWS = CT * N_QUANT
BT = 128
NBLK = TOK_W // BT

_mesh = plsc.VectorSubcoreMesh(core_axis_name="c", subcore_axis_name="s")


@functools.partial(
    pl.kernel,
    mesh=_mesh,
    out_type=jax.ShapeDtypeStruct((BATCH, SEQ, N_QUANT * CHUNK), jnp.float32),
    scratch_types=(
        [pltpu.VMEM((BT, N_QUANT), jnp.int32) for _ in range(2)]
        + [pltpu.VMEM((N_QUANT, 128), jnp.int32) for _ in range(2)]
        + [pltpu.VMEM((CROWS, CHUNK), jnp.float32) for _ in range(2)]
        + [pltpu.SemaphoreType.DMA((N_QUANT,)) for _ in range(2)]
        + [pltpu.SemaphoreType.DMA for _ in range(4)]
    ),
    compiler_params=pltpu.CompilerParams(needs_layout_passes=False),
)
def _emb_lookup(tbl_hbm, x_hbm, out_hbm, xsA, xsB, ipA, ipB, rowsA, rowsB,
                gsemA, gsemB, xsemA, xsemB, osemA, osemB):
    wid = lax.axis_index("s") * 2 + lax.axis_index("c")
    b = wid // 2
    t_base = (wid % 2) * TOK_W

    lanes = lax.iota(jnp.int32, 16)

    def stage_x(blk, xs, xsem):
        pltpu.async_copy(x_hbm.at[b, pl.ds(t_base + blk * BT, BT)], xs, xsem)

    def drain_x(xs, xsem):
        pltpu.make_async_copy(x_hbm.at[0, pl.ds(0, BT)], xs, xsem).wait()

    def build_perm(k, xs, ip):
        for g in range(CROWS // 16):
            iq = 16 * g // CT
            tl0 = (16 * g) % CT
            row = k * CT + tl0 + lanes
            col = jnp.full((16,), iq, jnp.int32)
            vals = plsc.load_gather(xs, [row, col])
            ip[iq, pl.ds(tl0, 16)] = vals

    def fire_gathers(ip, rows, gsem):
        for i in range(N_QUANT):
            pltpu.async_copy(
                tbl_hbm.at[i].at[ip.at[i, pl.ds(0, CT)]],
                rows.at[pl.ds(i * CT, CT)],
                gsem.at[i],
            )

    def drain_gather_fire_writebacks(c, rows, gsem, osem):
        t0 = t_base + c * CT
        for i in range(N_QUANT):
            pltpu.make_async_copy(
                tbl_hbm.at[0].at[pl.ds(0, CT)],
                rows.at[pl.ds(i * CT, CT)],
                gsem.at[i],
            ).wait()
            pltpu.async_copy(
                rows.at[pl.ds(i * CT, CT)],
                out_hbm.at[b, pl.ds(t0, CT), pl.ds(i * CHUNK, CHUNK)],
                osem,
            )

    def fire_writebacks(c, rows, osem):
        t0 = t_base + c * CT
        for i in range(N_QUANT):
            pltpu.async_copy(
                rows.at[pl.ds(i * CT, CT)],
                out_hbm.at[b, pl.ds(t0, CT), pl.ds(i * CHUNK, CHUNK)],
                osem,
            )

    def drain_writebacks(rows, osem):
        pltpu.make_async_copy(
            tbl_hbm.at[0].at[pl.ds(0, CROWS)], rows, osem
        ).wait()

    def process_block(blk, xs):
        for k in range(0, 4, 2):
            a = 4 * blk + k
            build_perm(k, xs, ipA)

            @pl.when(a > 0)
            def _():
                drain_writebacks(rowsA, osemA)
            fire_gathers(ipA, rowsA, gsemA)
            build_perm(k + 1, xs, ipB)

            @pl.when(a > 0)
            def _():
                drain_writebacks(rowsB, osemB)
            fire_gathers(ipB, rowsB, gsemB)
            drain_gather_fire_writebacks(a, rowsA, gsemA, osemA)
            drain_gather_fire_writebacks(a + 1, rowsB, gsemB, osemB)

    stage_x(0, xsA, xsemA)
    drain_x(xsA, xsemA)

    def body(g2, carry):
        blk = 2 * g2
        stage_x(blk + 1, xsB, xsemB)
        process_block(blk, xsA)
        drain_x(xsB, xsemB)

        @pl.when(g2 < NBLK // 2 - 1)
        def _():
            stage_x(blk + 2, xsA, xsemA)
        process_block(blk + 1, xsB)

        @pl.when(g2 < NBLK // 2 - 1)
        def _():
            drain_x(xsA, xsemA)
        return carry

    lax.fori_loop(0, NBLK // 2, body, 0)
    drain_writebacks(rowsA, osemA)
    drain_writebacks(rowsB, osemB)


def kernel(x, tables):
    return _emb_lookup(tables, x.astype(jnp.int32))
